# baseline (speedup 1.0000x reference)
import functools

import jax
import jax.numpy as jnp
from jax import lax
from jax.experimental import pallas as pl
from jax.experimental.pallas import tpu as pltpu
from jax.experimental.pallas import tpu_sc as plsc

_B, _C, _H, _W = 16, 32, 32, 32
_N = _B * _H * _W
_K = 8192
_TB = 128

_NW = 32
_RPW = _N // _NW
_CH = 128
_NCH = _RPW // _CH
_DP = 128


def _argmin_body(xb_ref, w_ref, xsq_ref, wsq_ref, idx_ref):
    dot = lax.dot_general(
        xb_ref[...], w_ref[...], (((1,), (1,)), ((), ())),
        preferred_element_type=jnp.float32)
    d2 = (xsq_ref[...] - 2.0 * dot) + wsq_ref[...]
    dists = jnp.sqrt(jnp.maximum(d2, 0.0))
    m = jnp.min(dists, axis=1, keepdims=True)
    ii = lax.broadcasted_iota(jnp.int32, (_TB, _K), 1)
    idx_ref[...] = jnp.min(
        jnp.where(dists == m, ii, jnp.int32(2147483647)),
        axis=1, keepdims=True)


def _sc_gather_body(tbl_hbm, idx_hbm, out_hbm, idx_v, rows_v, sem):
    wid = lax.axis_index("s") * 2 + lax.axis_index("c")
    pltpu.sync_copy(idx_hbm.at[wid], idx_v)
    for j in range(_NCH):
        pltpu.async_copy(tbl_hbm.at[idx_v.at[j]], rows_v.at[j], sem).wait()
    pltpu.sync_copy(rows_v, out_hbm.at[wid])


def _sc_gather(weight, idx3):
    mesh = plsc.VectorSubcoreMesh(core_axis_name="c", subcore_axis_name="s")
    run = functools.partial(
        pl.kernel,
        mesh=mesh,
        out_type=jax.ShapeDtypeStruct((_NW, _NCH, _CH, _DP), jnp.float32),
        scratch_types=[
            pltpu.VMEM((_NCH, _CH), jnp.int32),
            pltpu.VMEM((_NCH, _CH, _DP), jnp.float32),
            pltpu.SemaphoreType.DMA,
        ],
    )(_sc_gather_body)
    return run(weight, idx3)


def kernel(x, weight):
    b, c, h, w = x.shape
    flat = jnp.transpose(x, (0, 2, 3, 1)).reshape(-1, c)
    x_sq = jnp.sum(flat * flat, axis=1, keepdims=True)
    w_sq = jnp.sum(weight * weight, axis=1)[None, :]
    flat_bf = flat.astype(jnp.bfloat16)

    idx2 = pl.pallas_call(
        _argmin_body,
        grid=(_N // _TB,),
        in_specs=[
            pl.BlockSpec((_TB, c), lambda i: (i, 0)),
            pl.BlockSpec((_K, c), lambda i: (0, 0)),
            pl.BlockSpec((_TB, 1), lambda i: (i, 0)),
            pl.BlockSpec((1, _K), lambda i: (0, 0)),
        ],
        out_specs=pl.BlockSpec((_TB, 1), lambda i: (i, 0)),
        out_shape=jax.ShapeDtypeStruct((_N, 1), jnp.int32),
    )(flat_bf, weight, x_sq, w_sq)

    idx_flat = idx2.reshape(_N)
    w_pad = jnp.pad(weight, ((0, 0), (0, _DP - c)))
    q4 = _sc_gather(w_pad, idx_flat.reshape(_NW, _NCH, _CH))
    q_flat = q4.reshape(_N, _DP)[:, :c]

    quantized = jnp.transpose(q_flat.reshape(b, h, w, c), (0, 3, 1, 2))
    st = lax.stop_gradient(quantized - x) + x
    idxs = idx_flat.reshape(b, h, w)
    return (quantized, st, idxs)


# speedup vs baseline: 1.1795x; 1.1795x over previous
import functools

import jax
import jax.numpy as jnp
from jax import lax
from jax.experimental import pallas as pl
from jax.experimental.pallas import tpu as pltpu
from jax.experimental.pallas import tpu_sc as plsc

_B, _C, _H, _W = 16, 32, 32, 32
_N = _B * _H * _W
_K = 8192
_TB = 128

_NW = 32
_RPW = _N // _NW
_CH = 128
_NCH = _RPW // _CH
_DP = 128


def _argmin_body(xb_ref, w_ref, xsq_ref, wsq_ref, idx_ref):
    dot = lax.dot_general(
        xb_ref[...], w_ref[...], (((1,), (1,)), ((), ())),
        preferred_element_type=jnp.float32)
    d2 = (xsq_ref[...] - 2.0 * dot) + wsq_ref[...]
    d2c = jnp.maximum(d2, 0.0)
    m2 = jnp.min(d2c, axis=1, keepdims=True)
    s = jnp.sqrt(m2)
    mbits = lax.bitcast_convert_type(m2, jnp.int32)
    upper = m2
    alive = jnp.full(m2.shape, True)
    for t in range(1, 7):
        cand = lax.bitcast_convert_type(mbits + t, jnp.float32)
        alive = jnp.logical_and(alive, jnp.sqrt(cand) == s)
        upper = jnp.where(alive, cand, upper)
    ii = lax.broadcasted_iota(jnp.int32, (_TB, _K), 1)
    idx_ref[...] = jnp.min(
        jnp.where(d2c <= upper, ii, jnp.int32(2147483647)),
        axis=1, keepdims=True)


def _sc_gather_body(tbl_hbm, idx_hbm, out_hbm, idx_v, rows_v, sem):
    wid = lax.axis_index("s") * 2 + lax.axis_index("c")
    pltpu.sync_copy(idx_hbm.at[wid], idx_v)
    for j in range(_NCH):
        pltpu.async_copy(tbl_hbm.at[idx_v.at[j]], rows_v.at[j], sem).wait()
    pltpu.sync_copy(rows_v, out_hbm.at[wid])


def _sc_gather(weight, idx3):
    mesh = plsc.VectorSubcoreMesh(core_axis_name="c", subcore_axis_name="s")
    run = functools.partial(
        pl.kernel,
        mesh=mesh,
        out_type=jax.ShapeDtypeStruct((_NW, _NCH, _CH, _DP), jnp.float32),
        scratch_types=[
            pltpu.VMEM((_NCH, _CH), jnp.int32),
            pltpu.VMEM((_NCH, _CH, _DP), jnp.float32),
            pltpu.SemaphoreType.DMA,
        ],
    )(_sc_gather_body)
    return run(weight, idx3)


def kernel(x, weight):
    b, c, h, w = x.shape
    flat = jnp.transpose(x, (0, 2, 3, 1)).reshape(-1, c)
    x_sq = jnp.sum(flat * flat, axis=1, keepdims=True)
    w_sq = jnp.sum(weight * weight, axis=1)[None, :]
    flat_bf = flat.astype(jnp.bfloat16)

    idx2 = pl.pallas_call(
        _argmin_body,
        grid=(_N // _TB,),
        in_specs=[
            pl.BlockSpec((_TB, c), lambda i: (i, 0)),
            pl.BlockSpec((_K, c), lambda i: (0, 0)),
            pl.BlockSpec((_TB, 1), lambda i: (i, 0)),
            pl.BlockSpec((1, _K), lambda i: (0, 0)),
        ],
        out_specs=pl.BlockSpec((_TB, 1), lambda i: (i, 0)),
        out_shape=jax.ShapeDtypeStruct((_N, 1), jnp.int32),
    )(flat_bf, weight, x_sq, w_sq)

    idx_flat = idx2.reshape(_N)
    w_pad = jnp.pad(weight, ((0, 0), (0, _DP - c)))
    q4 = _sc_gather(w_pad, idx_flat.reshape(_NW, _NCH, _CH))
    q_flat = q4.reshape(_N, _DP)[:, :c]

    quantized = jnp.transpose(q_flat.reshape(b, h, w, c), (0, 3, 1, 2))
    st = lax.stop_gradient(quantized - x) + x
    idxs = idx_flat.reshape(b, h, w)
    return (quantized, st, idxs)
